# baseline (speedup 1.0000x reference)
import dataclasses

import jax
import jax.numpy as jnp
from jax import lax
from jax.experimental import pallas as pl
from jax.experimental.pallas import tpu as pltpu
from jax.experimental.pallas import tpu_sc as plsc

NN = 10000
DD = 128
EE = 320000
NPAD = 10240
K = 80
CPT = EE // 32 // K
RPT = NN // 16

_mesh = plsc.VectorSubcoreMesh(core_axis_name="c", subcore_axis_name="s")

_cp = pltpu.CompilerParams()
if "needs_layout_passes" in pltpu.CompilerParams.__dataclass_fields__:
    _cp = dataclasses.replace(_cp, needs_layout_passes=False)
_cp_flat = dataclasses.replace(_cp, use_tc_tiling_on_sc=False)


def _hist_body(dst_ref, zeros_ref, out_ref, hist_sh, idx_v, ones_v, zbuf_v):
    c = lax.axis_index("c")
    s = lax.axis_index("s")
    wid = c * 16 + s
    pltpu.sync_copy(zeros_ref, zbuf_v)
    pltpu.sync_copy(zbuf_v, hist_sh.at[pl.ds(s * 640, 640)])
    pltpu.sync_copy(dst_ref.at[wid], idx_v)
    for i in range(K // 16):
        ones_v[pl.ds(i * 16, 16)] = jnp.full((16,), 1.0, jnp.float32)
    plsc.subcore_barrier()

    @pl.loop(0, CPT)
    def _(j):
        pltpu.sync_copy(ones_v, hist_sh.at[idx_v.at[j]], add=True)

    plsc.subcore_barrier()
    pltpu.sync_copy(hist_sh.at[pl.ds(s * 640, 640)], zbuf_v)
    pltpu.sync_copy(zbuf_v, out_ref.at[pl.ds(c * NPAD + s * 640, 640)])


def _deg_hist(dst2d, zeros640):
    f = pl.kernel(
        _hist_body,
        out_type=jax.ShapeDtypeStruct((2 * NPAD,), jnp.float32),
        mesh=_mesh,
        scratch_types=[
            pltpu.VMEM_SHARED((NPAD,), jnp.float32),
            pltpu.VMEM((CPT, K), jnp.int32),
            pltpu.VMEM((K,), jnp.float32),
            pltpu.VMEM((640,), jnp.float32),
        ],
    )
    return f(dst2d, zeros640)


def _agg_body(y_ref, src_ref, dst_ref, zrows_ref, out_ref,
              agg_sh, sidx_v, didx_v, rows_a, rows_b, sa, sb, ssa, ssb):
    c = lax.axis_index("c")
    s = lax.axis_index("s")
    wid = c * 16 + s
    pltpu.sync_copy(zrows_ref, rows_a)
    for t in range(8):
        pltpu.sync_copy(rows_a, agg_sh.at[pl.ds(s * 640 + t * K, K)])
    pltpu.sync_copy(src_ref.at[pl.ds(wid * (CPT * K), CPT * K)], sidx_v)
    pltpu.sync_copy(dst_ref.at[wid], didx_v)
    plsc.subcore_barrier()

    pltpu.async_copy(y_ref.at[sidx_v.at[pl.ds(0, K)]], rows_a, sa)

    @pl.loop(0, CPT, step=2)
    def _(j):
        @pl.when(j + 1 < CPT)
        def _():
            pltpu.async_copy(y_ref.at[sidx_v.at[pl.ds((j + 1) * K, K)]], rows_b, sb)

        pltpu.make_async_copy(y_ref.at[sidx_v.at[pl.ds(j * K, K)]], rows_a, sa).wait()
        pltpu.sync_copy(rows_a, agg_sh.at[didx_v.at[j]], add=True)

        @pl.when(j + 2 < CPT)
        def _():
            pltpu.async_copy(y_ref.at[sidx_v.at[pl.ds((j + 2) * K, K)]], rows_a, sa)

        @pl.when(j + 1 < CPT)
        def _():
            pltpu.make_async_copy(y_ref.at[sidx_v.at[pl.ds((j + 1) * K, K)]], rows_b, sb).wait()
            pltpu.sync_copy(rows_b, agg_sh.at[didx_v.at[j + 1]], add=True)

    plsc.subcore_barrier()
    for t in range(8):
        pltpu.sync_copy(agg_sh.at[pl.ds(s * 640 + t * K, K)], rows_a)
        pltpu.sync_copy(rows_a, out_ref.at[c, pl.ds(s * 640 + t * K, K)])


def _agg(y, src1d, dst3d, zrows):
    f = pl.kernel(
        _agg_body,
        out_type=jax.ShapeDtypeStruct((2, NPAD, DD), jnp.float32),
        mesh=_mesh,
        scratch_types=[
            pltpu.VMEM_SHARED((NPAD, DD), jnp.float32),
            pltpu.VMEM((CPT * K,), jnp.int32),
            pltpu.VMEM((CPT, K), jnp.int32),
            pltpu.VMEM((K, DD), jnp.float32),
            pltpu.VMEM((K, DD), jnp.float32),
            pltpu.SemaphoreType.DMA,
            pltpu.SemaphoreType.DMA,
            pltpu.SemaphoreType.DMA,
            pltpu.SemaphoreType.DMA,
        ],
    )
    return f(y, src1d, dst3d, zrows)


def _pred_body(h_ref, i0_ref, i1_ref, out_ref,
               idx0_v, idx1_v, r0a, r1a, r0b, r1b, ob_v, sa, sb):
    c = lax.axis_index("c")
    s = lax.axis_index("s")
    wid = c * 16 + s
    pltpu.sync_copy(i0_ref.at[pl.ds(wid * (CPT * K), CPT * K)], idx0_v)
    pltpu.sync_copy(i1_ref.at[pl.ds(wid * (CPT * K), CPT * K)], idx1_v)

    lane = lax.iota(jnp.int32, 16)
    fzero = jnp.zeros((16,), jnp.float32)

    def issue(j, r0_v, r1_v, sem):
        pltpu.async_copy(h_ref.at[idx0_v.at[pl.ds(j * K, K)]], r0_v, sem)
        pltpu.async_copy(h_ref.at[idx1_v.at[pl.ds(j * K, K)]], r1_v, sem)

    def wait(j, r0_v, r1_v, sem):
        pltpu.make_async_copy(h_ref.at[idx0_v.at[pl.ds(j * K, K)]], r0_v, sem).wait()
        pltpu.make_async_copy(h_ref.at[idx1_v.at[pl.ds(j * K, K)]], r1_v, sem).wait()

    def compute(j, r0_v, r1_v):
        def tree_sum(vals):
            while len(vals) > 1:
                vals = [a + b for a, b in zip(vals[::2], vals[1::2])]
            return vals[0]

        for g in range(K // 16):
            terms = []
            for e in range(16):
                ei = g * 16 + e
                prods = []
                for t in range(4):
                    pp = r0_v[ei, pl.ds(t * 32, 32)] * r1_v[ei, pl.ds(t * 32, 32)]
                    p0, p1 = plsc.unpack(pp, format=plsc.PackFormat.INTERLEAVED)
                    prods.append(p0)
                    prods.append(p1)
                terms.append(jnp.where(lane == e, jnp.sum(tree_sum(prods)), fzero))
            ob_v[pl.ds(g * 16, 16)] = tree_sum(terms)
        pltpu.sync_copy(ob_v, out_ref.at[pl.ds(wid * (CPT * K) + j * K, K)])

    issue(0, r0a, r1a, sa)

    @pl.loop(0, CPT, step=2)
    def _(j):
        @pl.when(j + 1 < CPT)
        def _():
            issue(j + 1, r0b, r1b, sb)

        wait(j, r0a, r1a, sa)
        compute(j, r0a, r1a)

        @pl.when(j + 2 < CPT)
        def _():
            issue(j + 2, r0a, r1a, sa)

        @pl.when(j + 1 < CPT)
        def _():
            wait(j + 1, r0b, r1b, sb)
            compute(j + 1, r0b, r1b)


def _pred(h2, el0, el1):
    f = pl.kernel(
        _pred_body,
        out_type=jax.ShapeDtypeStruct((EE,), jnp.float32),
        mesh=_mesh,
        compiler_params=_cp_flat,
        scratch_types=[
            pltpu.VMEM((CPT * K,), jnp.int32),
            pltpu.VMEM((CPT * K,), jnp.int32),
            pltpu.VMEM((K, DD), jnp.bfloat16),
            pltpu.VMEM((K, DD), jnp.bfloat16),
            pltpu.VMEM((K, DD), jnp.bfloat16),
            pltpu.VMEM((K, DD), jnp.bfloat16),
            pltpu.VMEM((K,), jnp.float32),
            pltpu.SemaphoreType.DMA,
            pltpu.SemaphoreType.DMA,
        ],
    )
    return f(h2, el0, el1)


_GB = 10
_BR = NN // _GB


def _tc_mm_body(x_ref, w_ref, o_ref):
    o_ref[...] = jnp.dot(x_ref[...], w_ref[...],
                         preferred_element_type=jnp.float32)


def _matmul(x, w):
    return pl.pallas_call(
        _tc_mm_body,
        grid=(_GB,),
        in_specs=[pl.BlockSpec((_BR, DD), lambda i: (i, 0)),
                  pl.BlockSpec((DD, DD), lambda i: (0, 0))],
        out_specs=pl.BlockSpec((_BR, DD), lambda i: (i, 0)),
        out_shape=jax.ShapeDtypeStruct((NN, DD), jnp.float32),
    )(x, w)


def _tc_b_body(xw_ref, degt_ref, y_ref, dinv_ref):
    deg = degt_ref[:, 0:1] + degt_ref[:, 1:2] + 1.0
    dinv = lax.rsqrt(deg)
    y_ref[...] = xw_ref[...] * dinv
    dinv_ref[...] = dinv


def _stage_b(xw, degt):
    return pl.pallas_call(
        _tc_b_body,
        grid=(_GB,),
        in_specs=[pl.BlockSpec((_BR, DD), lambda i: (i, 0)),
                  pl.BlockSpec((_BR, 2), lambda i: (i, 0))],
        out_specs=(pl.BlockSpec((_BR, DD), lambda i: (i, 0)),
                   pl.BlockSpec((_BR, 1), lambda i: (i, 0))),
        out_shape=(jax.ShapeDtypeStruct((NN, DD), jnp.float32),
                   jax.ShapeDtypeStruct((NN, 1), jnp.float32)),
    )(xw, degt)


def _tc_d_body(p_ref, y1_ref, dinv_ref, b1_ref, w2_ref, y2_ref):
    agg = p_ref[0] + p_ref[1] + y1_ref[...]
    h = jnp.maximum(agg * dinv_ref[...] + b1_ref[...], 0.0)
    hw = jnp.dot(h, w2_ref[...], preferred_element_type=jnp.float32)
    y2_ref[...] = hw * dinv_ref[...]


def _stage_d(p, y1, dinv, b1r, w2):
    return pl.pallas_call(
        _tc_d_body,
        grid=(_GB,),
        in_specs=[pl.BlockSpec((2, _BR, DD), lambda i: (0, i, 0)),
                  pl.BlockSpec((_BR, DD), lambda i: (i, 0)),
                  pl.BlockSpec((_BR, 1), lambda i: (i, 0)),
                  pl.BlockSpec((1, DD), lambda i: (0, 0)),
                  pl.BlockSpec((DD, DD), lambda i: (0, 0))],
        out_specs=pl.BlockSpec((_BR, DD), lambda i: (i, 0)),
        out_shape=jax.ShapeDtypeStruct((NN, DD), jnp.float32),
    )(p, y1, dinv, b1r, w2)


def _tc_f_body(q_ref, y2_ref, dinv_ref, b2_ref, h2_ref):
    agg = q_ref[0] + q_ref[1] + y2_ref[...]
    h2_ref[...] = (agg * dinv_ref[...] + b2_ref[...]).astype(jnp.bfloat16)


def _stage_f(q, y2, dinv, b2r):
    return pl.pallas_call(
        _tc_f_body,
        grid=(_GB,),
        in_specs=[pl.BlockSpec((2, _BR, DD), lambda i: (0, i, 0)),
                  pl.BlockSpec((_BR, DD), lambda i: (i, 0)),
                  pl.BlockSpec((_BR, 1), lambda i: (i, 0)),
                  pl.BlockSpec((1, DD), lambda i: (0, 0))],
        out_specs=pl.BlockSpec((_BR, DD), lambda i: (i, 0)),
        out_shape=jax.ShapeDtypeStruct((NN, DD), jnp.bfloat16),
    )(q, y2, dinv, b2r)


def kernel(node_feature, edge_index, edge_label_index, W1, b1, W2, b2):
    src1d = edge_index[0]
    dst3d = edge_index[1].reshape(32, CPT, K)
    el0 = edge_label_index[0]
    el1 = edge_label_index[1]
    zeros640 = jnp.zeros((640,), jnp.float32)
    zrows = jnp.zeros((K, DD), jnp.float32)
    b1r = b1.reshape(1, DD)
    b2r = b2.reshape(1, DD)

    xw = _matmul(node_feature, W1)
    hist = _deg_hist(dst3d, zeros640)
    degt = hist.reshape(2, NPAD)[:, :NN].T
    y1, dinv = _stage_b(xw, degt)
    p = _agg(y1, src1d, dst3d, zrows)[:, :NN]
    y2 = _stage_d(p, y1, dinv, b1r, W2)
    q = _agg(y2, src1d, dst3d, zrows)[:, :NN]
    h2 = _stage_f(q, y2, dinv, b2r)
    return _pred(h2, el0, el1)


# speedup vs baseline: 1.0158x; 1.0158x over previous
import dataclasses

import jax
import jax.numpy as jnp
from jax import lax
from jax.experimental import pallas as pl
from jax.experimental.pallas import tpu as pltpu
from jax.experimental.pallas import tpu_sc as plsc

NN = 10000
DD = 128
EE = 320000
NPAD = 10240
K = 80
CPT = EE // 32 // K
RPT = NN // 16

_mesh = plsc.VectorSubcoreMesh(core_axis_name="c", subcore_axis_name="s")

_cp = pltpu.CompilerParams()
if "needs_layout_passes" in pltpu.CompilerParams.__dataclass_fields__:
    _cp = dataclasses.replace(_cp, needs_layout_passes=False)
_cp_flat = dataclasses.replace(_cp, use_tc_tiling_on_sc=False)


def _hist_body(dst_ref, zeros_ref, out_ref, hist_sh, idx_v, ones_v, zbuf_v):
    c = lax.axis_index("c")
    s = lax.axis_index("s")
    wid = c * 16 + s
    pltpu.sync_copy(zeros_ref, zbuf_v)
    pltpu.sync_copy(zbuf_v, hist_sh.at[pl.ds(s * 640, 640)])
    pltpu.sync_copy(dst_ref.at[wid], idx_v)
    for i in range(K // 16):
        ones_v[pl.ds(i * 16, 16)] = jnp.full((16,), 1.0, jnp.float32)
    plsc.subcore_barrier()

    @pl.loop(0, CPT)
    def _(j):
        pltpu.sync_copy(ones_v, hist_sh.at[idx_v.at[j]], add=True)

    plsc.subcore_barrier()
    pltpu.sync_copy(hist_sh.at[pl.ds(s * 640, 640)], zbuf_v)
    pltpu.sync_copy(zbuf_v, out_ref.at[pl.ds(c * NPAD + s * 640, 640)])


def _deg_hist(dst2d, zeros640):
    f = pl.kernel(
        _hist_body,
        out_type=jax.ShapeDtypeStruct((2 * NPAD,), jnp.float32),
        mesh=_mesh,
        scratch_types=[
            pltpu.VMEM_SHARED((NPAD,), jnp.float32),
            pltpu.VMEM((CPT, K), jnp.int32),
            pltpu.VMEM((K,), jnp.float32),
            pltpu.VMEM((640,), jnp.float32),
        ],
    )
    return f(dst2d, zeros640)


def _agg_body(y_ref, src_ref, dst_ref, zrows_ref, out_ref,
              agg_sh, sidx_v, didx_v, rows_a, rows_b, sa, sb, ssa, ssb):
    c = lax.axis_index("c")
    s = lax.axis_index("s")
    wid = c * 16 + s
    pltpu.sync_copy(zrows_ref, rows_a)
    for t in range(8):
        pltpu.sync_copy(rows_a, agg_sh.at[pl.ds(s * 640 + t * K, K)])
    pltpu.sync_copy(src_ref.at[pl.ds(wid * (CPT * K), CPT * K)], sidx_v)
    pltpu.sync_copy(dst_ref.at[wid], didx_v)
    plsc.subcore_barrier()

    pltpu.async_copy(y_ref.at[sidx_v.at[pl.ds(0, K)]], rows_a, sa)

    @pl.loop(0, CPT, step=2)
    def _(j):
        @pl.when(j + 1 < CPT)
        def _():
            pltpu.async_copy(y_ref.at[sidx_v.at[pl.ds((j + 1) * K, K)]], rows_b, sb)

        pltpu.make_async_copy(y_ref.at[sidx_v.at[pl.ds(j * K, K)]], rows_a, sa).wait()
        pltpu.sync_copy(rows_a, agg_sh.at[didx_v.at[j]], add=True)

        @pl.when(j + 2 < CPT)
        def _():
            pltpu.async_copy(y_ref.at[sidx_v.at[pl.ds((j + 2) * K, K)]], rows_a, sa)

        @pl.when(j + 1 < CPT)
        def _():
            pltpu.make_async_copy(y_ref.at[sidx_v.at[pl.ds((j + 1) * K, K)]], rows_b, sb).wait()
            pltpu.sync_copy(rows_b, agg_sh.at[didx_v.at[j + 1]], add=True)

    plsc.subcore_barrier()
    for t in range(8):
        pltpu.sync_copy(agg_sh.at[pl.ds(s * 640 + t * K, K)], rows_a)
        pltpu.sync_copy(rows_a, out_ref.at[c, pl.ds(s * 640 + t * K, K)])


def _agg(y, src1d, dst3d, zrows):
    f = pl.kernel(
        _agg_body,
        out_type=jax.ShapeDtypeStruct((2, NPAD, DD), jnp.float32),
        mesh=_mesh,
        scratch_types=[
            pltpu.VMEM_SHARED((NPAD, DD), jnp.float32),
            pltpu.VMEM((CPT * K,), jnp.int32),
            pltpu.VMEM((CPT, K), jnp.int32),
            pltpu.VMEM((K, DD), jnp.float32),
            pltpu.VMEM((K, DD), jnp.float32),
            pltpu.SemaphoreType.DMA,
            pltpu.SemaphoreType.DMA,
            pltpu.SemaphoreType.DMA,
            pltpu.SemaphoreType.DMA,
        ],
    )
    return f(y, src1d, dst3d, zrows)


def _pred_body(h_ref, i0_ref, i1_ref, out_ref,
               idx0_v, idx1_v, r0a, r1a, r0b, r1b, ob_v, sa, sb):
    c = lax.axis_index("c")
    s = lax.axis_index("s")
    wid = c * 16 + s
    pltpu.sync_copy(i0_ref.at[pl.ds(wid * (CPT * K), CPT * K)], idx0_v)
    pltpu.sync_copy(i1_ref.at[pl.ds(wid * (CPT * K), CPT * K)], idx1_v)

    lane = lax.iota(jnp.int32, 16)
    fzero = jnp.zeros((16,), jnp.float32)

    def issue(j, r0_v, r1_v, sem):
        pltpu.async_copy(h_ref.at[idx0_v.at[pl.ds(j * K, K)]], r0_v, sem)
        pltpu.async_copy(h_ref.at[idx1_v.at[pl.ds(j * K, K)]], r1_v, sem)

    def wait(j, r0_v, r1_v, sem):
        pltpu.make_async_copy(h_ref.at[idx0_v.at[pl.ds(j * K, K)]], r0_v, sem).wait()
        pltpu.make_async_copy(h_ref.at[idx1_v.at[pl.ds(j * K, K)]], r1_v, sem).wait()

    def compute(j, r0_v, r1_v):
        def tree_sum(vals):
            while len(vals) > 1:
                vals = [a + b for a, b in zip(vals[::2], vals[1::2])]
            return vals[0]

        for g in range(K // 16):
            terms = []
            for e in range(16):
                ei = g * 16 + e
                prods = []
                for t in range(4):
                    pp = r0_v[ei, pl.ds(t * 32, 32)] * r1_v[ei, pl.ds(t * 32, 32)]
                    p0, p1 = plsc.unpack(pp, format=plsc.PackFormat.INTERLEAVED)
                    prods.append(p0)
                    prods.append(p1)
                terms.append(jnp.where(lane == e, jnp.sum(tree_sum(prods)), fzero))
            ob_v[pl.ds(g * 16, 16)] = tree_sum(terms)
        pltpu.sync_copy(ob_v, out_ref.at[pl.ds(wid * (CPT * K) + j * K, K)])

    issue(0, r0a, r1a, sa)

    @pl.loop(0, CPT, step=2)
    def _(j):
        @pl.when(j + 1 < CPT)
        def _():
            issue(j + 1, r0b, r1b, sb)

        wait(j, r0a, r1a, sa)
        compute(j, r0a, r1a)

        @pl.when(j + 2 < CPT)
        def _():
            issue(j + 2, r0a, r1a, sa)

        @pl.when(j + 1 < CPT)
        def _():
            wait(j + 1, r0b, r1b, sb)
            compute(j + 1, r0b, r1b)


def _pred(h2, el0, el1):
    f = pl.kernel(
        _pred_body,
        out_type=jax.ShapeDtypeStruct((EE,), jnp.float32),
        mesh=_mesh,
        compiler_params=_cp_flat,
        scratch_types=[
            pltpu.VMEM((CPT * K,), jnp.int32),
            pltpu.VMEM((CPT * K,), jnp.int32),
            pltpu.VMEM((K, DD), jnp.bfloat16),
            pltpu.VMEM((K, DD), jnp.bfloat16),
            pltpu.VMEM((K, DD), jnp.bfloat16),
            pltpu.VMEM((K, DD), jnp.bfloat16),
            pltpu.VMEM((K,), jnp.float32),
            pltpu.SemaphoreType.DMA,
            pltpu.SemaphoreType.DMA,
        ],
    )
    return f(h2, el0, el1)


def _tc_b_body(x_ref, w1_ref, degt_ref, y_ref, dinv_ref):
    deg = degt_ref[:, 0:1] + degt_ref[:, 1:2] + 1.0
    dinv = lax.rsqrt(deg)
    xw = jnp.dot(x_ref[...], w1_ref[...], preferred_element_type=jnp.float32)
    y_ref[...] = xw * dinv
    dinv_ref[...] = dinv


def _stage_b(x, w1, degt):
    return pl.pallas_call(
        _tc_b_body,
        out_shape=(jax.ShapeDtypeStruct((NN, DD), jnp.float32),
                   jax.ShapeDtypeStruct((NN, 1), jnp.float32)),
    )(x, w1, degt)


def _tc_d_body(p_ref, y1_ref, dinv_ref, b1_ref, w2_ref, y2_ref):
    agg = p_ref[0] + p_ref[1] + y1_ref[...]
    h = jnp.maximum(agg * dinv_ref[...] + b1_ref[...], 0.0)
    hw = jnp.dot(h, w2_ref[...], preferred_element_type=jnp.float32)
    y2_ref[...] = hw * dinv_ref[...]


def _stage_d(p, y1, dinv, b1r, w2):
    return pl.pallas_call(
        _tc_d_body,
        out_shape=jax.ShapeDtypeStruct((NN, DD), jnp.float32),
    )(p, y1, dinv, b1r, w2)


def _tc_f_body(q_ref, y2_ref, dinv_ref, b2_ref, h2_ref):
    agg = q_ref[0] + q_ref[1] + y2_ref[...]
    h2_ref[...] = (agg * dinv_ref[...] + b2_ref[...]).astype(jnp.bfloat16)


def _stage_f(q, y2, dinv, b2r):
    return pl.pallas_call(
        _tc_f_body,
        out_shape=jax.ShapeDtypeStruct((NN, DD), jnp.bfloat16),
    )(q, y2, dinv, b2r)


def kernel(node_feature, edge_index, edge_label_index, W1, b1, W2, b2):
    src1d = edge_index[0]
    dst3d = edge_index[1].reshape(32, CPT, K)
    el0 = edge_label_index[0]
    el1 = edge_label_index[1]
    zeros640 = jnp.zeros((640,), jnp.float32)
    zrows = jnp.zeros((K, DD), jnp.float32)
    b1r = b1.reshape(1, DD)
    b2r = b2.reshape(1, DD)

    hist = _deg_hist(dst3d, zeros640)
    degt = hist.reshape(2, NPAD)[:, :NN].T
    y1, dinv = _stage_b(node_feature, W1, degt)
    p = _agg(y1, src1d, dst3d, zrows)[:, :NN]
    y2 = _stage_d(p, y1, dinv, b1r, W2)
    q = _agg(y2, src1d, dst3d, zrows)[:, :NN]
    h2 = _stage_f(q, y2, dinv, b2r)
    return _pred(h2, el0, el1)
